# baseline (speedup 1.0000x reference)
import functools

import jax
import jax.numpy as jnp
from jax import lax
from jax.experimental import pallas as pl
from jax.experimental.pallas import tpu as pltpu

ALPHA = 0.2
N = 2048
F = 16
ROW_BLK = 128


def _fused_gat_body(x_ref, w1_ref, a1_ref, w2_ref, a2_ref, o_ref, wh_s, h_s):
    def layer(xin, W, a, write_out):
        wh = jnp.dot(xin, W, preferred_element_type=jnp.float32)
        wh_s[...] = wh
        f1 = jnp.dot(wh, a[:F, :], preferred_element_type=jnp.float32)
        f2c = jnp.dot(wh, a[F:, :], preferred_element_type=jnp.float32)
        f2r = lax.dot_general(
            a[F:, :], wh,
            dimension_numbers=(((0,), (1,)), ((), ())),
            preferred_element_type=jnp.float32,
        )
        u = jnp.exp(f1)
        v = jnp.exp(ALPHA * f1)
        g = jnp.exp(f2c)
        gh = jnp.exp(ALPHA * f2c)
        ones = jnp.ones((N, 1), jnp.float32)
        who = jnp.concatenate([wh, ones], axis=1)
        qf = jnp.concatenate([who * g, who * gh], axis=1)
        q = qf.astype(jnp.bfloat16)
        tn = jnp.sum(qf[:, F + 1:], axis=0, keepdims=True)
        f2rb = f2r.astype(jnp.bfloat16)
        nf1b = (-f1).astype(jnp.bfloat16)
        one_b = jnp.bfloat16(1.0)
        zero_b = jnp.bfloat16(0.0)
        for j in range(N // ROW_BLK):
            sl = slice(j * ROW_BLK, (j + 1) * ROW_BLK)
            mask = jnp.where(f2rb >= nf1b[sl, :], one_b, zero_b)
            m = jnp.dot(mask, q, preferred_element_type=jnp.float32)
            mp = m[:, :F + 1]
            mn = tn - m[:, F + 1:]
            num = u[sl, :] * mp[:, :F] + v[sl, :] * mn[:, :F]
            den = u[sl, :] * mp[:, F:] + v[sl, :] * mn[:, F:]
            h = num / den
            write_out(j, jnp.where(h > 0, h, jnp.exp(h) - 1.0))

    w1 = w1_ref[...]
    a1 = a1_ref[...]
    w2 = w2_ref[...]
    a2 = a2_ref[...]
    for b in range(8):
        def write_h(j, val):
            h_s[pl.ds(j * ROW_BLK, ROW_BLK), :] = val

        def write_o(j, val, b=b):
            o_ref[b, pl.ds(j * ROW_BLK, ROW_BLK), :] = val

        layer(x_ref[b], w1, a1, write_h)
        layer(h_s[...], w2, a2, write_o)


@jax.jit
def kernel(x, W1, a1, W2, a2):
    B = x.shape[0]
    return pl.pallas_call(
        _fused_gat_body,
        out_shape=jax.ShapeDtypeStruct((B, N, F), jnp.float32),
        scratch_shapes=[
            pltpu.VMEM((N, F), jnp.float32),
            pltpu.VMEM((N, F), jnp.float32),
        ],
    )(x, W1, a1, W2, a2)


# speedup vs baseline: 1.0460x; 1.0460x over previous
import functools

import jax
import jax.numpy as jnp
from jax import lax
from jax.experimental import pallas as pl
from jax.experimental.pallas import tpu as pltpu

ALPHA = 0.2
N = 2048
F = 16
ROW_BLK = 256


def _fused_gat_body(x_ref, w1_ref, a1_ref, w2_ref, a2_ref, o_ref, wh_s, h_s):
    def layer(xin, W, a, write_out):
        wh = jnp.dot(xin, W, preferred_element_type=jnp.float32)
        wh_s[...] = wh
        f1 = jnp.dot(wh, a[:F, :], preferred_element_type=jnp.float32)
        f2c = jnp.dot(wh, a[F:, :], preferred_element_type=jnp.float32)
        f2r = lax.dot_general(
            a[F:, :], wh,
            dimension_numbers=(((0,), (1,)), ((), ())),
            preferred_element_type=jnp.float32,
        )
        u = jnp.exp(f1)
        v = jnp.exp(ALPHA * f1)
        g = jnp.exp(f2c)
        gh = jnp.exp(ALPHA * f2c)
        ones = jnp.ones((N, 1), jnp.float32)
        who = jnp.concatenate([wh, ones], axis=1)
        qf = jnp.concatenate([who * g, who * gh], axis=1)
        q = qf.astype(jnp.bfloat16)
        tn = jnp.sum(qf[:, F + 1:], axis=0, keepdims=True)
        f2rb = f2r.astype(jnp.bfloat16)
        nf1b = (-f1).astype(jnp.bfloat16)
        one_b = jnp.bfloat16(1.0)
        zero_b = jnp.bfloat16(0.0)
        for j in range(N // ROW_BLK):
            sl = slice(j * ROW_BLK, (j + 1) * ROW_BLK)
            mask = jnp.where(f2rb >= nf1b[sl, :], one_b, zero_b)
            m = jnp.dot(mask, q, preferred_element_type=jnp.float32)
            mp = m[:, :F + 1]
            mn = tn - m[:, F + 1:]
            num = u[sl, :] * mp[:, :F] + v[sl, :] * mn[:, :F]
            den = u[sl, :] * mp[:, F:] + v[sl, :] * mn[:, F:]
            h = num / den
            write_out(j, jnp.where(h > 0, h, jnp.exp(h) - 1.0))

    w1 = w1_ref[...]
    a1 = a1_ref[...]
    w2 = w2_ref[...]
    a2 = a2_ref[...]
    for b in range(8):
        def write_h(j, val):
            h_s[pl.ds(j * ROW_BLK, ROW_BLK), :] = val

        def write_o(j, val, b=b):
            o_ref[b, pl.ds(j * ROW_BLK, ROW_BLK), :] = val

        layer(x_ref[b], w1, a1, write_h)
        layer(h_s[...], w2, a2, write_o)


@jax.jit
def kernel(x, W1, a1, W2, a2):
    B = x.shape[0]
    return pl.pallas_call(
        _fused_gat_body,
        out_shape=jax.ShapeDtypeStruct((B, N, F), jnp.float32),
        scratch_shapes=[
            pltpu.VMEM((N, F), jnp.float32),
            pltpu.VMEM((N, F), jnp.float32),
        ],
    )(x, W1, a1, W2, a2)


# speedup vs baseline: 1.0611x; 1.0144x over previous
import functools

import jax
import jax.numpy as jnp
from jax import lax
from jax.experimental import pallas as pl
from jax.experimental.pallas import tpu as pltpu

ALPHA = 0.2
N = 2048
F = 16
FP = 32
ROW_BLK = 256


def _fused_gat_body(x_ref, w1_ref, a1_ref, w2_ref, a2_ref, o_ref,
                    who_s, h_s):
    who_s[...] = jnp.zeros((N, FP), jnp.float32)
    who_s[:, F:F + 1] = jnp.ones((N, 1), jnp.float32)

    w1 = w1_ref[...]
    a1 = a1_ref[...]
    w2 = w2_ref[...]
    a2 = a2_ref[...]

    def layer(xin, W, a, write_out):
        wh = jnp.dot(xin, W, preferred_element_type=jnp.float32)
        who_s[:, :F] = wh
        f1 = jnp.dot(wh, a[:F, :], preferred_element_type=jnp.float32)
        f2c = jnp.dot(wh, a[F:, :], preferred_element_type=jnp.float32)
        f2r = lax.dot_general(
            a[F:, :], wh,
            dimension_numbers=(((0,), (1,)), ((), ())),
            preferred_element_type=jnp.float32,
        )
        u = jnp.exp(f1)
        v = jnp.exp(ALPHA * f1)
        g = jnp.exp(f2c)
        gh = jnp.exp(ALPHA * f2c)
        who = who_s[...]
        qn = who * gh
        q = jnp.concatenate([who * g, qn], axis=1).astype(jnp.bfloat16)
        tn = jnp.sum(qn, axis=0, keepdims=True)
        f2rb = f2r.astype(jnp.bfloat16)
        nf1b = (-f1).astype(jnp.bfloat16)
        one_b = jnp.bfloat16(1.0)
        zero_b = jnp.bfloat16(0.0)
        for j in range(N // ROW_BLK):
            sl = slice(j * ROW_BLK, (j + 1) * ROW_BLK)
            mask = jnp.where(f2rb >= nf1b[sl, :], one_b, zero_b)
            m = jnp.dot(mask, q, preferred_element_type=jnp.float32)
            nd = u[sl, :] * m[:, :FP] + v[sl, :] * (tn - m[:, FP:])
            h = nd[:, :F] / nd[:, F:F + 1]
            write_out(j, jnp.where(h > 0, h, jnp.exp(h) - 1.0))

    for b in range(8):
        def write_h(j, val):
            h_s[pl.ds(j * ROW_BLK, ROW_BLK), :] = val

        def write_o(j, val, b=b):
            o_ref[b, pl.ds(j * ROW_BLK, ROW_BLK), :] = val

        layer(x_ref[b], w1, a1, write_h)
        layer(h_s[...], w2, a2, write_o)


@jax.jit
def kernel(x, W1, a1, W2, a2):
    B = x.shape[0]
    return pl.pallas_call(
        _fused_gat_body,
        out_shape=jax.ShapeDtypeStruct((B, N, F), jnp.float32),
        scratch_shapes=[
            pltpu.VMEM((N, FP), jnp.float32),
            pltpu.VMEM((N, F), jnp.float32),
        ],
    )(x, W1, a1, W2, a2)


# speedup vs baseline: 1.0625x; 1.0014x over previous
import jax
import jax.numpy as jnp
from jax import lax
from jax.experimental import pallas as pl
from jax.experimental.pallas import tpu as pltpu

ALPHA = 0.2
N = 2048
F = 16
FP = 32
ROW_BLK = 256


def _fused_gat_body(x_ref, w1_ref, a1_ref, w2_ref, a2_ref, o_ref,
                    who_s, h_s):
    who_s[...] = jnp.zeros((N, FP), jnp.float32)
    who_s[:, F:F + 1] = jnp.ones((N, 1), jnp.float32)

    w1 = w1_ref[...]
    a1 = a1_ref[...]
    w2 = w2_ref[...]
    a2 = a2_ref[...]

    def layer(xin, W, a, write_out):
        wh = jnp.dot(xin, W, preferred_element_type=jnp.float32)
        who_s[:, :F] = wh
        f1 = jnp.dot(wh, a[:F, :], preferred_element_type=jnp.float32)
        f2c = jnp.dot(wh, a[F:, :], preferred_element_type=jnp.float32)
        f2r = lax.dot_general(
            a[F:, :], wh,
            dimension_numbers=(((0,), (1,)), ((), ())),
            preferred_element_type=jnp.float32,
        )
        u = jnp.exp(f1)
        v = jnp.exp(ALPHA * f1)
        g = jnp.exp(f2c)
        gh = jnp.exp(ALPHA * f2c)
        who = who_s[...]
        qn = who * gh
        q = jnp.concatenate([who * g, qn], axis=1).astype(jnp.bfloat16)
        tn = jnp.sum(qn, axis=0, keepdims=True)
        f2rb = f2r.astype(jnp.bfloat16)
        nf1b = (-f1).astype(jnp.bfloat16)
        one_b = jnp.bfloat16(1.0)
        zero_b = jnp.bfloat16(0.0)
        for j in range(N // ROW_BLK):
            sl = slice(j * ROW_BLK, (j + 1) * ROW_BLK)
            mask = jnp.where(f2rb >= nf1b[sl, :], one_b, zero_b)
            m = jnp.dot(mask, q, preferred_element_type=jnp.float32)
            nd = u[sl, :] * m[:, :FP] + v[sl, :] * (tn - m[:, FP:])
            h = nd[:, :F] / nd[:, F:F + 1]
            write_out(j, jnp.where(h > 0, h, jnp.exp(h) - 1.0))

    for b in range(8):
        def write_h(j, val):
            h_s[pl.ds(j * ROW_BLK, ROW_BLK), :] = val

        def write_o(j, val, b=b):
            o_ref[b, pl.ds(j * ROW_BLK, ROW_BLK), :] = val

        layer(x_ref[b], w1, a1, write_h)
        layer(h_s[...], w2, a2, write_o)


@jax.jit
def kernel(x, W1, a1, W2, a2):
    B = x.shape[0]
    return pl.pallas_call(
        _fused_gat_body,
        out_shape=jax.ShapeDtypeStruct((B, N, F), jnp.float32),
        scratch_shapes=[
            pltpu.VMEM((N, FP), jnp.float32),
            pltpu.VMEM((N, F), jnp.float32),
        ],
    )(x, W1, a1, W2, a2)
